# Initial kernel scaffold; baseline (speedup 1.0000x reference)
import jax
import jax.numpy as jnp
from jax.experimental import pallas as pl


def kernel(x_user, x_product, x_category, x_style, x_brand, x_color, edge_similar_to, edge_belongs_to, edge_has_style, edge_prefers, edge_viewed, proj_W_user, proj_b_user, proj_W_product, proj_b_product, proj_W_category, proj_b_category, proj_W_style, proj_b_style, proj_W_brand, proj_b_brand, proj_W_color, proj_b_color, c1_similar_to_W, c1_similar_to_as, c1_similar_to_ad, c1_similar_to_b, c1_belongs_to_W, c1_belongs_to_as, c1_belongs_to_ad, c1_belongs_to_b, c1_has_style_W, c1_has_style_as, c1_has_style_ad, c1_has_style_b, c1_prefers_W, c1_prefers_as, c1_prefers_ad, c1_prefers_b, c1_viewed_W, c1_viewed_as, c1_viewed_ad, c1_viewed_b, c2_similar_to_W, c2_similar_to_as, c2_similar_to_ad, c2_similar_to_b, c2_belongs_to_W, c2_belongs_to_as, c2_belongs_to_ad, c2_belongs_to_b, c2_prefers_W, c2_prefers_as, c2_prefers_ad, c2_prefers_b, out_W_product, out_b_product, out_W_category, out_b_category):
    raise NotImplementedError("write your pallas kernel here")


# speedup vs baseline: 1.0838x; 1.0838x over previous
import functools

import jax
import jax.numpy as jnp
from jax.experimental import pallas as pl

HID, OUT, HEADS = 64, 32, 4


def _proj_body(x_ref, w_ref, b_ref, o_ref):
    o_ref[...] = jax.nn.relu(x_ref[...] @ w_ref[...] + b_ref[...])


def _proj(x, W, b, blk):
    n = x.shape[0]
    return pl.pallas_call(
        _proj_body,
        grid=(n // blk,),
        in_specs=[
            pl.BlockSpec((blk, x.shape[1]), lambda i: (i, 0)),
            pl.BlockSpec((x.shape[1], W.shape[1]), lambda i: (0, 0)),
            pl.BlockSpec((1, W.shape[1]), lambda i: (0, 0)),
        ],
        out_specs=pl.BlockSpec((blk, W.shape[1]), lambda i: (i, 0)),
        out_shape=jax.ShapeDtypeStruct((n, W.shape[1]), jnp.float32),
    )(x, W, b.reshape(1, -1))


def _gat(x_src, x_dst, edge, W, att_s, att_d, b, heads, outc, self_loops):
    ns, nd = x_src.shape[0], x_dst.shape[0]
    hs = (x_src @ W).reshape(ns, heads, outc)
    hd = (x_dst @ W).reshape(nd, heads, outc)
    a_s = (hs * att_s).sum(-1)
    a_d = (hd * att_d).sum(-1)
    es, ed = edge[0], edge[1]
    if self_loops:
        loop = jnp.arange(nd, dtype=es.dtype)
        es = jnp.concatenate([es, loop])
        ed = jnp.concatenate([ed, loop])
    alpha = jax.nn.leaky_relu(a_s[es] + a_d[ed], 0.2)
    ex = jnp.exp(alpha)
    den = jax.ops.segment_sum(ex, ed, num_segments=nd)
    num = jax.ops.segment_sum(hs[es] * ex[:, :, None], ed, num_segments=nd)
    out = num / (den[:, :, None] + 1e-16)
    return out.mean(axis=1) + b


def kernel(x_user, x_product, x_category, x_style, x_brand, x_color, edge_similar_to, edge_belongs_to, edge_has_style, edge_prefers, edge_viewed, proj_W_user, proj_b_user, proj_W_product, proj_b_product, proj_W_category, proj_b_category, proj_W_style, proj_b_style, proj_W_brand, proj_b_brand, proj_W_color, proj_b_color, c1_similar_to_W, c1_similar_to_as, c1_similar_to_ad, c1_similar_to_b, c1_belongs_to_W, c1_belongs_to_as, c1_belongs_to_ad, c1_belongs_to_b, c1_has_style_W, c1_has_style_as, c1_has_style_ad, c1_has_style_b, c1_prefers_W, c1_prefers_as, c1_prefers_ad, c1_prefers_b, c1_viewed_W, c1_viewed_as, c1_viewed_ad, c1_viewed_b, c2_similar_to_W, c2_similar_to_as, c2_similar_to_ad, c2_similar_to_b, c2_belongs_to_W, c2_belongs_to_as, c2_belongs_to_ad, c2_belongs_to_b, c2_prefers_W, c2_prefers_as, c2_prefers_ad, c2_prefers_b, out_W_product, out_b_product, out_W_category, out_b_category):
    h_user = _proj(x_user, proj_W_user, proj_b_user, 2000)
    h_product = _proj(x_product, proj_W_product, proj_b_product, 2000)
    h_category = _proj(x_category, proj_W_category, proj_b_category, 1000)

    o_sim = _gat(h_product, h_product, edge_similar_to, c1_similar_to_W,
                 c1_similar_to_as, c1_similar_to_ad, c1_similar_to_b, HEADS, HID, True)
    o_bel = _gat(h_product, h_category, edge_belongs_to, c1_belongs_to_W,
                 c1_belongs_to_as, c1_belongs_to_ad, c1_belongs_to_b, HEADS, HID, False)
    o_pre = _gat(h_user, h_product, edge_prefers, c1_prefers_W,
                 c1_prefers_as, c1_prefers_ad, c1_prefers_b, HEADS, HID, False)
    o_vie = _gat(h_user, h_product, edge_viewed, c1_viewed_W,
                 c1_viewed_as, c1_viewed_ad, c1_viewed_b, HEADS, HID, False)
    h1_product = jax.nn.relu((o_sim + o_pre + o_vie) / 3.0)
    h1_category = jax.nn.relu(o_bel)

    h2_product = _gat(h1_product, h1_product, edge_similar_to, c2_similar_to_W,
                      c2_similar_to_as, c2_similar_to_ad, c2_similar_to_b, 1, OUT, True)
    h2_category = _gat(h1_product, h1_category, edge_belongs_to, c2_belongs_to_W,
                       c2_belongs_to_as, c2_belongs_to_ad, c2_belongs_to_b, 1, OUT, False)

    out_p = h2_product @ out_W_product + out_b_product
    out_c = h2_category @ out_W_category + out_b_category
    return (out_p, out_c)


# speedup vs baseline: 1.2416x; 1.1456x over previous
import functools

import jax
import jax.numpy as jnp
from jax.experimental import pallas as pl

HID, OUT, HEADS = 64, 32, 4
EPS = 1e-16


def _pre_body(x_ref, w_ref, b_ref, w1_ref, w2_ref, aw_ref, o1_ref, o2_ref, a_ref):
    h = jax.nn.relu(x_ref[...] @ w_ref[...] + b_ref[...])
    o1_ref[...] = h @ w1_ref[...]
    o2_ref[...] = h @ w2_ref[...]
    a_ref[...] = h @ aw_ref[...]


def _pre(x, W, b, W1, W2, Aw, blk):
    n = x.shape[0]
    return pl.pallas_call(
        _pre_body,
        grid=(n // blk,),
        in_specs=[
            pl.BlockSpec((blk, x.shape[1]), lambda i: (i, 0)),
            pl.BlockSpec(W.shape, lambda i: (0, 0)),
            pl.BlockSpec((1, HID), lambda i: (0, 0)),
            pl.BlockSpec(W1.shape, lambda i: (0, 0)),
            pl.BlockSpec(W2.shape, lambda i: (0, 0)),
            pl.BlockSpec(Aw.shape, lambda i: (0, 0)),
        ],
        out_specs=[
            pl.BlockSpec((blk, W1.shape[1]), lambda i: (i, 0)),
            pl.BlockSpec((blk, W2.shape[1]), lambda i: (i, 0)),
            pl.BlockSpec((blk, Aw.shape[1]), lambda i: (i, 0)),
        ],
        out_shape=[
            jax.ShapeDtypeStruct((n, W1.shape[1]), jnp.float32),
            jax.ShapeDtypeStruct((n, W2.shape[1]), jnp.float32),
            jax.ShapeDtypeStruct((n, Aw.shape[1]), jnp.float32),
        ],
    )(x, W, b.reshape(1, -1), W1, W2, Aw)


def _leaky(x):
    return jnp.where(x >= 0, x, 0.2 * x)


def _mid_p_body(acs_ref, dns_ref, ass_ref, ads_ref, hs_ref,
                acp_ref, dnp_ref, acv_ref, dnv_ref,
                bs_ref, bp_ref, bv_ref, w2s_ref, w2b_ref, a2w_ref,
                hs2s_ref, hs2b_ref, a2_ref):
    exs = jnp.exp(_leaky(ass_ref[:, :4] + ads_ref[:, :4]))
    o_sim = jnp.zeros_like(bs_ref[...])
    o_pre = jnp.zeros_like(o_sim)
    o_vie = jnp.zeros_like(o_sim)
    for h in range(HEADS):
        e = exs[:, h:h + 1]
        o_sim = o_sim + (acs_ref[h] + e * hs_ref[:, h, :]) / (
            dns_ref[:, h:h + 1] + e + EPS)
        o_pre = o_pre + acp_ref[h] / (dnp_ref[:, h:h + 1] + EPS)
        o_vie = o_vie + acv_ref[h] / (dnv_ref[:, h:h + 1] + EPS)
    o_sim = o_sim / HEADS + bs_ref[...]
    o_pre = o_pre / HEADS + bp_ref[...]
    o_vie = o_vie / HEADS + bv_ref[...]
    h1 = jax.nn.relu((o_sim + o_pre + o_vie) / 3.0)
    hs2s_ref[...] = h1 @ w2s_ref[...]
    hs2b_ref[...] = h1 @ w2b_ref[...]
    a2_ref[...] = h1 @ a2w_ref[...]


def _mid_product(acs, dns, ass, ads, hs3, acp, dnp, acv, dnv,
                 bs, bp, bv, W2s, W2b, A2w, nd, blk):
    vec = lambda i: (i, 0)
    pl3 = lambda i: (0, i, 0)
    return pl.pallas_call(
        _mid_p_body,
        grid=(nd // blk,),
        in_specs=[
            pl.BlockSpec((HEADS, blk, HID), pl3),
            pl.BlockSpec((blk, 16), vec),
            pl.BlockSpec((blk, 16), vec),
            pl.BlockSpec((blk, 16), vec),
            pl.BlockSpec((blk, HEADS, HID), lambda i: (i, 0, 0)),
            pl.BlockSpec((HEADS, blk, HID), pl3),
            pl.BlockSpec((blk, 16), vec),
            pl.BlockSpec((HEADS, blk, HID), pl3),
            pl.BlockSpec((blk, 16), vec),
            pl.BlockSpec((1, HID), lambda i: (0, 0)),
            pl.BlockSpec((1, HID), lambda i: (0, 0)),
            pl.BlockSpec((1, HID), lambda i: (0, 0)),
            pl.BlockSpec((HID, OUT), lambda i: (0, 0)),
            pl.BlockSpec((HID, OUT), lambda i: (0, 0)),
            pl.BlockSpec((HID, 48), lambda i: (0, 0)),
        ],
        out_specs=[
            pl.BlockSpec((blk, OUT), vec),
            pl.BlockSpec((blk, OUT), vec),
            pl.BlockSpec((blk, 48), vec),
        ],
        out_shape=[
            jax.ShapeDtypeStruct((nd, OUT), jnp.float32),
            jax.ShapeDtypeStruct((nd, OUT), jnp.float32),
            jax.ShapeDtypeStruct((nd, 48), jnp.float32),
        ],
    )(acs, dns, ass, ads, hs3, acp, dnp, acv, dnv,
      bs.reshape(1, -1), bp.reshape(1, -1), bv.reshape(1, -1), W2s, W2b, A2w)


def _mid_c_body(ac_ref, dn_ref, b_ref, a2w_ref, a2_ref):
    o = jnp.zeros_like(b_ref[...])
    for h in range(HEADS):
        o = o + ac_ref[h] / (dn_ref[:, h:h + 1] + EPS)
    h1 = jax.nn.relu(o / HEADS + b_ref[...])
    a2_ref[...] = h1 @ a2w_ref[...]


def _mid_category(ac, dn, b, A2w, nd):
    return pl.pallas_call(
        _mid_c_body,
        grid=(1,),
        in_specs=[
            pl.BlockSpec((HEADS, nd, HID), lambda i: (0, 0, 0)),
            pl.BlockSpec((nd, 16), lambda i: (0, 0)),
            pl.BlockSpec((1, HID), lambda i: (0, 0)),
            pl.BlockSpec((HID, 16), lambda i: (0, 0)),
        ],
        out_specs=pl.BlockSpec((nd, 16), lambda i: (0, 0)),
        out_shape=jax.ShapeDtypeStruct((nd, 16), jnp.float32),
    )(ac, dn, b.reshape(1, -1), A2w)


def _fin_p_body(ac_ref, dn_ref, as_ref, ad_ref, hs2_ref, b_ref, w_ref, bo_ref, o_ref):
    exs = jnp.exp(_leaky(as_ref[:, 0:1] + ad_ref[:, 0:1]))
    h2 = (ac_ref[...] + exs * hs2_ref[...]) / (dn_ref[:, 0:1] + exs + EPS) + b_ref[...]
    o_ref[...] = h2 @ w_ref[...] + bo_ref[...]


def _final_product(ac, dn, as2, ad2, hs2, b2, Wo, bo, nd, blk):
    vec = lambda i: (i, 0)
    return pl.pallas_call(
        _fin_p_body,
        grid=(nd // blk,),
        in_specs=[
            pl.BlockSpec((blk, OUT), vec),
            pl.BlockSpec((blk, 16), vec),
            pl.BlockSpec((blk, 16), vec),
            pl.BlockSpec((blk, 16), vec),
            pl.BlockSpec((blk, OUT), vec),
            pl.BlockSpec((1, OUT), lambda i: (0, 0)),
            pl.BlockSpec((OUT, OUT), lambda i: (0, 0)),
            pl.BlockSpec((1, OUT), lambda i: (0, 0)),
        ],
        out_specs=pl.BlockSpec((blk, OUT), vec),
        out_shape=jax.ShapeDtypeStruct((nd, OUT), jnp.float32),
    )(ac, dn, as2, ad2, hs2, b2.reshape(1, -1), Wo, bo.reshape(1, -1))


def _fin_c_body(ac_ref, dn_ref, b_ref, w_ref, bo_ref, o_ref):
    h2 = ac_ref[...] / (dn_ref[:, 0:1] + EPS) + b_ref[...]
    o_ref[...] = h2 @ w_ref[...] + bo_ref[...]


def _final_category(ac, dn, b2, Wo, bo, nd):
    return pl.pallas_call(
        _fin_c_body,
        grid=(1,),
        in_specs=[
            pl.BlockSpec((nd, OUT), lambda i: (0, 0)),
            pl.BlockSpec((nd, 16), lambda i: (0, 0)),
            pl.BlockSpec((1, OUT), lambda i: (0, 0)),
            pl.BlockSpec((OUT, OUT), lambda i: (0, 0)),
            pl.BlockSpec((1, OUT), lambda i: (0, 0)),
        ],
        out_specs=pl.BlockSpec((nd, OUT), lambda i: (0, 0)),
        out_shape=jax.ShapeDtypeStruct((nd, OUT), jnp.float32),
    )(ac, dn, b2.reshape(1, -1), Wo, bo.reshape(1, -1))


def _veff(W, att, heads, outc):
    W3 = W.reshape(W.shape[0], heads, outc)
    V = (W3 * att[0][None, :, :]).sum(-1)
    return jnp.pad(V, ((0, 0), (0, 16 - heads)))


def _agg(es, ed, as_tab, ad_tab, hs3, heads, nd):
    ex = jnp.exp(_leaky(as_tab[es, :heads] + ad_tab[ed, :heads]))
    den = jax.ops.segment_sum(ex, ed, num_segments=nd)
    num = jax.ops.segment_sum(hs3[es] * ex[:, :, None], ed, num_segments=nd)
    accum = jnp.transpose(num, (1, 0, 2))
    den16 = jnp.pad(den, ((0, 0), (0, 16 - heads)))
    return accum, den16


def kernel(x_user, x_product, x_category, x_style, x_brand, x_color, edge_similar_to, edge_belongs_to, edge_has_style, edge_prefers, edge_viewed, proj_W_user, proj_b_user, proj_W_product, proj_b_product, proj_W_category, proj_b_category, proj_W_style, proj_b_style, proj_W_brand, proj_b_brand, proj_W_color, proj_b_color, c1_similar_to_W, c1_similar_to_as, c1_similar_to_ad, c1_similar_to_b, c1_belongs_to_W, c1_belongs_to_as, c1_belongs_to_ad, c1_belongs_to_b, c1_has_style_W, c1_has_style_as, c1_has_style_ad, c1_has_style_b, c1_prefers_W, c1_prefers_as, c1_prefers_ad, c1_prefers_b, c1_viewed_W, c1_viewed_as, c1_viewed_ad, c1_viewed_b, c2_similar_to_W, c2_similar_to_as, c2_similar_to_ad, c2_similar_to_b, c2_belongs_to_W, c2_belongs_to_as, c2_belongs_to_ad, c2_belongs_to_b, c2_prefers_W, c2_prefers_as, c2_prefers_ad, c2_prefers_b, out_W_product, out_b_product, out_W_category, out_b_category):
    np_, nu, nc = x_product.shape[0], x_user.shape[0], x_category.shape[0]

    Aw_p = jnp.concatenate([
        _veff(c1_similar_to_W, c1_similar_to_as, HEADS, HID),
        _veff(c1_similar_to_W, c1_similar_to_ad, HEADS, HID),
        _veff(c1_belongs_to_W, c1_belongs_to_as, HEADS, HID),
        _veff(c1_prefers_W, c1_prefers_ad, HEADS, HID),
        _veff(c1_viewed_W, c1_viewed_ad, HEADS, HID),
    ], axis=1)
    Aw_u = jnp.concatenate([
        _veff(c1_prefers_W, c1_prefers_as, HEADS, HID),
        _veff(c1_viewed_W, c1_viewed_as, HEADS, HID),
    ], axis=1)
    A2w_p = jnp.concatenate([
        _veff(c2_similar_to_W, c2_similar_to_as, 1, OUT),
        _veff(c2_similar_to_W, c2_similar_to_ad, 1, OUT),
        _veff(c2_belongs_to_W, c2_belongs_to_as, 1, OUT),
    ], axis=1)
    A2w_c = _veff(c2_belongs_to_W, c2_belongs_to_ad, 1, OUT)

    hs_sim, hs_bel, A_p = _pre(x_product, proj_W_product, proj_b_product,
                               c1_similar_to_W, c1_belongs_to_W, Aw_p, 2000)
    hs_pre, hs_vie, A_u = _pre(x_user, proj_W_user, proj_b_user,
                               c1_prefers_W, c1_viewed_W, Aw_u, 2000)
    Aw_c = _veff(c1_belongs_to_W, c1_belongs_to_ad, HEADS, HID)
    _, _, A_c = _pre(x_category, proj_W_category, proj_b_category,
                     Aw_c, Aw_c, Aw_c, 1000)

    as_sim, ad_sim = A_p[:, 0:16], A_p[:, 16:32]
    as_bel = A_p[:, 32:48]
    ad_pre, ad_vie = A_p[:, 48:64], A_p[:, 64:80]
    as_pre, as_vie = A_u[:, 0:16], A_u[:, 16:32]
    ad_bel = A_c

    hs_sim3 = hs_sim.reshape(np_, HEADS, HID)
    hs_bel3 = hs_bel.reshape(np_, HEADS, HID)
    hs_pre3 = hs_pre.reshape(nu, HEADS, HID)
    hs_vie3 = hs_vie.reshape(nu, HEADS, HID)

    acs, dns = _agg(edge_similar_to[0], edge_similar_to[1], as_sim, ad_sim,
                    hs_sim3, HEADS, np_)
    acb, dnb = _agg(edge_belongs_to[0], edge_belongs_to[1], as_bel, ad_bel,
                    hs_bel3, HEADS, nc)
    acp, dnp = _agg(edge_prefers[0], edge_prefers[1], as_pre, ad_pre,
                    hs_pre3, HEADS, np_)
    acv, dnv = _agg(edge_viewed[0], edge_viewed[1], as_vie, ad_vie,
                    hs_vie3, HEADS, np_)

    hs2s, hs2b, A2_p = _mid_product(acs, dns, as_sim, ad_sim, hs_sim3,
                                    acp, dnp, acv, dnv,
                                    c1_similar_to_b, c1_prefers_b, c1_viewed_b,
                                    c2_similar_to_W, c2_belongs_to_W, A2w_p,
                                    np_, 2000)
    ad2_bel = _mid_category(acb, dnb, c1_belongs_to_b, A2w_c, nc)
    as2_sim, ad2_sim = A2_p[:, 0:16], A2_p[:, 16:32]
    as2_bel = A2_p[:, 32:48]

    ac2s, dn2s = _agg(edge_similar_to[0], edge_similar_to[1], as2_sim, ad2_sim,
                      hs2s[:, None, :], 1, np_)
    ac2b, dn2b = _agg(edge_belongs_to[0], edge_belongs_to[1], as2_bel, ad2_bel,
                      hs2b[:, None, :], 1, nc)

    out_p = _final_product(ac2s[0], dn2s, as2_sim, ad2_sim, hs2s,
                           c2_similar_to_b, out_W_product, out_b_product, np_, 2000)
    out_c = _final_category(ac2b[0], dn2b, c2_belongs_to_b,
                            out_W_category, out_b_category, nc)
    return (out_p, out_c)


# speedup vs baseline: 4.2514x; 3.4242x over previous
import functools
import math

import jax
import jax.numpy as jnp
from jax import lax
from jax.experimental import pallas as pl
from jax.experimental.pallas import tpu as pltpu
from jax.experimental.pallas import tpu_sc as plsc

HID, OUT, HEADS = 64, 32, 4
EPS = 1e-16


def _leaky(x):
    return jnp.where(x >= 0, x, 0.2 * x)


def _pre_body(x_ref, w_ref, b_ref, w1_ref, w2_ref, aw_ref,
              e1a_ref, e1b_ref, e2a_ref, e2b_ref, o1b_ref, exsl_ref, tab_ref):
    h = jax.nn.relu(x_ref[...] @ w_ref[...] + b_ref[...])
    o1 = h @ w1_ref[...]
    o1b_ref[...] = o1
    if w1_ref.shape[1] >= 256:
        e1a_ref[...] = o1[:, 0:128]
        e1b_ref[...] = o1[:, 128:256]
        o2 = h @ w2_ref[...]
        e2a_ref[...] = o2[:, 0:128]
        e2b_ref[...] = o2[:, 128:256]
    else:
        z = jnp.zeros_like(e1a_ref)
        e1a_ref[...] = z
        e1b_ref[...] = z
        e2a_ref[...] = z
        e2b_ref[...] = z
    a = h @ aw_ref[...]
    aw = aw_ref.shape[1]
    tab_ref[:, 0:aw] = a
    tab_ref[:, aw:128] = jnp.zeros((a.shape[0], 128 - aw), jnp.float32)
    if aw >= 32:
        exsl_ref[...] = jnp.exp(_leaky(a[:, 0:16] + a[:, 16:32]))
    else:
        exsl_ref[...] = jnp.zeros_like(exsl_ref)


def _pre(x, W, b, W1, W2, Aw, blk):
    n = x.shape[0]
    return pl.pallas_call(
        _pre_body,
        grid=(n // blk,),
        in_specs=[
            pl.BlockSpec((blk, x.shape[1]), lambda i: (i, 0)),
            pl.BlockSpec(W.shape, lambda i: (0, 0)),
            pl.BlockSpec((1, HID), lambda i: (0, 0)),
            pl.BlockSpec(W1.shape, lambda i: (0, 0)),
            pl.BlockSpec(W2.shape, lambda i: (0, 0)),
            pl.BlockSpec(Aw.shape, lambda i: (0, 0)),
        ],
        out_specs=[
            pl.BlockSpec((blk, 128), lambda i: (i, 0)),
            pl.BlockSpec((blk, 128), lambda i: (i, 0)),
            pl.BlockSpec((blk, 128), lambda i: (i, 0)),
            pl.BlockSpec((blk, 128), lambda i: (i, 0)),
            pl.BlockSpec((blk, W1.shape[1]), lambda i: (i, 0)),
            pl.BlockSpec((blk, 16), lambda i: (i, 0)),
            pl.BlockSpec((blk, 128), lambda i: (i, 0)),
        ],
        out_shape=[
            jax.ShapeDtypeStruct((n, 128), jnp.float32),
            jax.ShapeDtypeStruct((n, 128), jnp.float32),
            jax.ShapeDtypeStruct((n, 128), jnp.float32),
            jax.ShapeDtypeStruct((n, 128), jnp.float32),
            jax.ShapeDtypeStruct((n, W1.shape[1]), jnp.float32),
            jax.ShapeDtypeStruct((n, 16), jnp.float32),
            jax.ShapeDtypeStruct((n + 8, 128), jnp.float32),
        ],
    )(x, W, b.reshape(1, -1), W1, W2, Aw)


def _mid_p_body(acs_ref, dns_ref, exsl_ref, hs_ref,
                acp_ref, dnp_ref, acv_ref, dnv_ref,
                bs_ref, bp_ref, bv_ref, w2s_ref, w2b_ref, a2w_ref,
                hs2sb_ref, exsl2_ref, t2_ref, a2t_ref):
    exs = exsl_ref[:, :4]
    o_sim = jnp.zeros_like(bs_ref[...])
    o_pre = jnp.zeros_like(o_sim)
    o_vie = jnp.zeros_like(o_sim)
    for h in range(HEADS):
        p, q = h // 2, h % 2
        e = exs[:, h:h + 1]
        o_sim = o_sim + (acs_ref[p][:, q * 64:q * 64 + 64] + e * hs_ref[:, h, :]) / (
            dns_ref[:, h:h + 1] + e + EPS)
        o_pre = o_pre + acp_ref[p][:, q * 64:q * 64 + 64] / (dnp_ref[:, h:h + 1] + EPS)
        o_vie = o_vie + acv_ref[p][:, q * 64:q * 64 + 64] / (dnv_ref[:, h:h + 1] + EPS)
    o_sim = o_sim / HEADS + bs_ref[...]
    o_pre = o_pre / HEADS + bp_ref[...]
    o_vie = o_vie / HEADS + bv_ref[...]
    h1 = jax.nn.relu((o_sim + o_pre + o_vie) / 3.0)
    hs2s = h1 @ w2s_ref[...]
    hs2b = h1 @ w2b_ref[...]
    hs2sb_ref[...] = hs2s
    n = hs2s.shape[0]
    t2_ref[:, 0:OUT] = hs2s
    t2_ref[:, OUT:2 * OUT] = hs2b
    t2_ref[:, 2 * OUT:128] = jnp.zeros((n, 128 - 2 * OUT), jnp.float32)
    a2 = h1 @ a2w_ref[...]
    a2t_ref[:, 0:48] = a2
    a2t_ref[:, 48:128] = jnp.zeros((n, 80), jnp.float32)
    exsl2_ref[...] = jnp.exp(_leaky(a2[:, 0:16] + a2[:, 16:32]))


def _mid_product(acs, dns, exsl, hs3, acp, dnp, acv, dnv,
                 bs, bp, bv, W2s, W2b, A2w, nd, blk):
    vec = lambda i: (i, 0)
    pl3 = lambda i: (0, i, 0)
    return pl.pallas_call(
        _mid_p_body,
        grid=(nd // blk,),
        in_specs=[
            pl.BlockSpec((2, blk, 128), pl3),
            pl.BlockSpec((blk, 128), vec),
            pl.BlockSpec((blk, 16), vec),
            pl.BlockSpec((blk, HEADS, HID), lambda i: (i, 0, 0)),
            pl.BlockSpec((2, blk, 128), pl3),
            pl.BlockSpec((blk, 128), vec),
            pl.BlockSpec((2, blk, 128), pl3),
            pl.BlockSpec((blk, 128), vec),
            pl.BlockSpec((1, HID), lambda i: (0, 0)),
            pl.BlockSpec((1, HID), lambda i: (0, 0)),
            pl.BlockSpec((1, HID), lambda i: (0, 0)),
            pl.BlockSpec((HID, OUT), lambda i: (0, 0)),
            pl.BlockSpec((HID, OUT), lambda i: (0, 0)),
            pl.BlockSpec((HID, 48), lambda i: (0, 0)),
        ],
        out_specs=[
            pl.BlockSpec((blk, OUT), vec),
            pl.BlockSpec((blk, 16), vec),
            pl.BlockSpec((blk, 128), vec),
            pl.BlockSpec((blk, 128), vec),
        ],
        out_shape=[
            jax.ShapeDtypeStruct((nd, OUT), jnp.float32),
            jax.ShapeDtypeStruct((nd, 16), jnp.float32),
            jax.ShapeDtypeStruct((nd + 8, 128), jnp.float32),
            jax.ShapeDtypeStruct((nd + 8, 128), jnp.float32),
        ],
    )(acs, dns, exsl, hs3, acp, dnp, acv, dnv,
      bs.reshape(1, -1), bp.reshape(1, -1), bv.reshape(1, -1), W2s, W2b, A2w)


def _mid_c_body(ac_ref, dn_ref, b_ref, a2w_ref, a2t_ref):
    o = jnp.zeros_like(b_ref[...])
    for h in range(HEADS):
        p, q = h // 2, h % 2
        o = o + ac_ref[p][:, q * 64:q * 64 + 64] / (dn_ref[:, h:h + 1] + EPS)
    h1 = jax.nn.relu(o / HEADS + b_ref[...])
    a2 = h1 @ a2w_ref[...]
    n = a2.shape[0]
    a2t_ref[:, 0:16] = a2
    a2t_ref[:, 16:128] = jnp.zeros((n, 112), jnp.float32)


def _mid_category(ac, dn, b, A2w, nd):
    return pl.pallas_call(
        _mid_c_body,
        grid=(1,),
        in_specs=[
            pl.BlockSpec((2, nd, 128), lambda i: (0, 0, 0)),
            pl.BlockSpec((nd, 128), lambda i: (0, 0)),
            pl.BlockSpec((1, HID), lambda i: (0, 0)),
            pl.BlockSpec((HID, 16), lambda i: (0, 0)),
        ],
        out_specs=pl.BlockSpec((nd, 128), lambda i: (0, 0)),
        out_shape=jax.ShapeDtypeStruct((nd + 8, 128), jnp.float32),
    )(ac, dn, b.reshape(1, -1), A2w)


def _fin_p_body(ac_ref, dn_ref, exsl2_ref, hs2_ref, b_ref, w_ref, bo_ref, o_ref):
    exs = exsl2_ref[:, 0:1]
    h2 = (ac_ref[:, 0:OUT] + exs * hs2_ref[...]) / (
        dn_ref[:, 0:1] + exs + EPS) + b_ref[...]
    o_ref[...] = h2 @ w_ref[...] + bo_ref[...]


def _final_product(ac, dn, exsl2, hs2, b2, Wo, bo, nd, blk):
    vec = lambda i: (i, 0)
    return pl.pallas_call(
        _fin_p_body,
        grid=(nd // blk,),
        in_specs=[
            pl.BlockSpec((blk, 128), vec),
            pl.BlockSpec((blk, 128), vec),
            pl.BlockSpec((blk, 16), vec),
            pl.BlockSpec((blk, OUT), vec),
            pl.BlockSpec((1, OUT), lambda i: (0, 0)),
            pl.BlockSpec((OUT, OUT), lambda i: (0, 0)),
            pl.BlockSpec((1, OUT), lambda i: (0, 0)),
        ],
        out_specs=pl.BlockSpec((blk, OUT), vec),
        out_shape=jax.ShapeDtypeStruct((nd, OUT), jnp.float32),
    )(ac, dn, exsl2, hs2, b2.reshape(1, -1), Wo, bo.reshape(1, -1))


def _fin_c_body(ac_ref, dn_ref, b_ref, w_ref, bo_ref, o_ref):
    h2 = ac_ref[:, OUT:2 * OUT] / (dn_ref[:, 0:1] + EPS) + b_ref[...]
    o_ref[...] = h2 @ w_ref[...] + bo_ref[...]


def _final_category(ac, dn, b2, Wo, bo, nd):
    return pl.pallas_call(
        _fin_c_body,
        grid=(1,),
        in_specs=[
            pl.BlockSpec((nd, 128), lambda i: (0, 0)),
            pl.BlockSpec((nd, 128), lambda i: (0, 0)),
            pl.BlockSpec((1, OUT), lambda i: (0, 0)),
            pl.BlockSpec((OUT, OUT), lambda i: (0, 0)),
            pl.BlockSpec((1, OUT), lambda i: (0, 0)),
        ],
        out_specs=pl.BlockSpec((nd, OUT), lambda i: (0, 0)),
        out_shape=jax.ShapeDtypeStruct((nd, OUT), jnp.float32),
    )(ac, dn, b2.reshape(1, -1), Wo, bo.reshape(1, -1))


def _bcast16(vec, idx):
    dn = lax.GatherDimensionNumbers(
        offset_dims=(), collapsed_slice_dims=(0,), start_index_map=(0,))
    return lax.gather(vec, idx[:, None], dn, (1,),
                      mode=lax.GatherScatterMode.PROMISE_IN_BOUNDS)


_K = 128
_MESH = dict(core_axis_name="c", subcore_axis_name="s")


def _region(n_rows, max_chunk=104):
    r = math.ceil(n_rows / 16)
    m = math.ceil(r / max_chunk)
    chunk = math.ceil(r / (m * 8)) * 8
    return 16 * chunk * m, chunk, m


def _zero_fill(zb, chunk, w):
    zvec = jnp.zeros((16,), jnp.float32)

    def zfill(i, _):
        for cc in range(w // 16):
            zb[i, pl.ds(cc * 16, 16)] = zvec
        return 0
    lax.fori_loop(0, chunk, zfill, 0)


@functools.lru_cache(maxsize=None)
def _sc_stage_a(E_pad, ns, nd, CS, CD):
    n_ch = E_pad // (32 * _K)
    mesh = plsc.VectorSubcoreMesh(**_MESH)

    @functools.partial(
        pl.kernel, mesh=mesh,
        out_type=jax.ShapeDtypeStruct((E_pad, 16), jnp.float32),
        scratch_types=[
            pltpu.VMEM((_K,), jnp.int32),
            pltpu.VMEM((_K,), jnp.int32),
            pltpu.VMEM((_K, 128), jnp.float32),
            pltpu.VMEM((_K, 128), jnp.float32),
            pltpu.VMEM((_K, 16), jnp.float32),
            pltpu.SemaphoreType.DMA,
            pltpu.SemaphoreType.DMA,
        ])
    def kern(es_hbm, ed_hbm, tabs_hbm, tabd_hbm, ex_hbm,
             es_v, ed_v, asr, adr, exr, sem1, sem2):
        c = lax.axis_index("c")
        s = lax.axis_index("s")
        wid = c * 16 + s

        def body(t, _):
            goff = (wid * n_ch + t) * _K
            pltpu.sync_copy(es_hbm.at[pl.ds(goff, _K)], es_v)
            pltpu.sync_copy(ed_hbm.at[pl.ds(goff, _K)], ed_v)
            pltpu.async_copy(tabs_hbm.at[es_v], asr, sem1).wait()
            pltpu.async_copy(tabd_hbm.at[ed_v], adr, sem2).wait()
            for kk in range(_K):
                a = asr[kk, pl.ds(CS, 16)] + adr[kk, pl.ds(CD, 16)]
                exr[kk, :] = jnp.exp(jnp.where(a >= 0, a, 0.2 * a))
            pltpu.sync_copy(exr, ex_hbm.at[pl.ds(goff, _K)])
            return 0
        lax.fori_loop(0, n_ch, body, 0)
    return kern


@functools.lru_cache(maxsize=None)
def _sc_den(E_pad, nd, half):
    if half:
        DQ, chunk, m = _region(-(-nd // 8))
        n_ch = E_pad // (16 * _K)
        out_ty = jax.ShapeDtypeStruct((8 * DQ, 128), jnp.float32)
        n_pass = 4
    else:
        DQ, chunk, m = _region(nd + 1)
        n_ch = E_pad // (32 * _K)
        out_ty = jax.ShapeDtypeStruct((2, DQ, 128), jnp.float32)
        n_pass = 1
    mesh = plsc.VectorSubcoreMesh(**_MESH)

    @functools.partial(
        pl.kernel, mesh=mesh,
        out_type=out_ty,
        scratch_types=[
            pltpu.VMEM((_K,), jnp.int32),
            pltpu.VMEM((_K,), jnp.int32),
            pltpu.VMEM((_K, 16), jnp.float32),
            pltpu.VMEM((_K, 128), jnp.float32),
            pltpu.VMEM((chunk, 128), jnp.float32),
            pltpu.VMEM((chunk, 128), jnp.float32),
            pltpu.VMEM_SHARED((DQ + 8, 128), jnp.float32),
        ])
    def kern(es_hbm, ed_hbm, ex_hbm, den_hbm,
             ed_v, idx_v, exc, rows, zb, bo, den_sh):
        c = lax.axis_index("c")
        s = lax.axis_index("s")
        _zero_fill(zb, chunk, 128)
        zvec = jnp.zeros((16,), jnp.float32)

        def rfill(i, _):
            for cc in range(1, 8):
                rows[i, pl.ds(cc * 16, 16)] = zvec
            return 0
        lax.fori_loop(0, _K, rfill, 0)

        def pass_body(ps, _):
            base = (c * 4 + ps) * DQ if half else 0
            for j in range(m):
                pltpu.sync_copy(zb, den_sh.at[pl.ds((s * m + j) * chunk, chunk)])
            plsc.subcore_barrier()

            def body(t, _):
                if half:
                    goff = (s * n_ch + t) * _K
                else:
                    goff = ((c * 16 + s) * n_ch + t) * _K
                pltpu.sync_copy(ed_hbm.at[pl.ds(goff, _K)], ed_v)
                pltpu.sync_copy(ex_hbm.at[pl.ds(goff, _K)], exc)
                for kk in range(_K):
                    rows[kk, pl.ds(0, 16)] = exc[kk, :]
                if half:
                    for k16 in range(_K // 16):
                        sl = pl.ds(k16 * 16, 16)
                        dv = ed_v[sl] - base
                        ok = (dv >= 0) & (dv < DQ)
                        idx_v[sl] = jnp.where(ok, dv, DQ)
                    pltpu.sync_copy(rows, den_sh.at[idx_v], add=True)
                else:
                    pltpu.sync_copy(rows, den_sh.at[ed_v], add=True)
                return 0
            lax.fori_loop(0, n_ch, body, 0)
            plsc.subcore_barrier()

            for j in range(m):
                off = (s * m + j) * chunk
                pltpu.sync_copy(den_sh.at[pl.ds(off, chunk)], bo)
                if half:
                    pltpu.sync_copy(bo, den_hbm.at[pl.ds(base + off, chunk)])
                else:
                    pltpu.sync_copy(bo, den_hbm.at[c, pl.ds(off, chunk)])
            plsc.subcore_barrier()
            return 0
        lax.fori_loop(0, n_pass, pass_body, 0)
    return kern


@functools.lru_cache(maxsize=None)
def _sc_b1(E_pad, ns2, nd, half):
    if half:
        QR, chunk, m = _region(-(-nd // 8))
        n_ch = E_pad // (16 * _K)
        out_ty = jax.ShapeDtypeStruct((2, 8 * QR, 128), jnp.float32)
        n_pass = 8
    else:
        QR, chunk, m = _region(nd + 1)
        n_ch = E_pad // (32 * _K)
        out_ty = jax.ShapeDtypeStruct((2, 2, QR, 128), jnp.float32)
        n_pass = 2
    mesh = plsc.VectorSubcoreMesh(**_MESH)

    @functools.partial(
        pl.kernel, mesh=mesh,
        out_type=out_ty,
        scratch_types=[
            pltpu.VMEM((_K,), jnp.int32),
            pltpu.VMEM((_K,), jnp.int32),
            pltpu.VMEM((_K,), jnp.int32),
            pltpu.VMEM((_K, 16), jnp.float32),
            pltpu.VMEM((_K, 128), jnp.float32),
            pltpu.VMEM((chunk, 128), jnp.float32),
            pltpu.VMEM((chunk, 128), jnp.float32),
            pltpu.VMEM_SHARED((QR + 8, 128), jnp.float32),
            pltpu.SemaphoreType.DMA,
        ])
    def kern(es_hbm, ed_hbm, ex_hbm, hs_hbm, ac_hbm,
             es_v, ed_v, idx_v, exc, rows, zb, bo, acc_sh, sem1):
        c = lax.axis_index("c")
        s = lax.axis_index("s")
        _zero_fill(zb, chunk, 128)

        def pass_body(ps, _):
            if half:
                rg = ps // 2
                p = ps % 2
                base = (c * 4 + rg) * QR
            else:
                p = ps
                base = 0
            for j in range(m):
                pltpu.sync_copy(zb, acc_sh.at[pl.ds((s * m + j) * chunk, chunk)])
            plsc.subcore_barrier()

            def body(t, _):
                if half:
                    goff = (s * n_ch + t) * _K
                else:
                    goff = ((c * 16 + s) * n_ch + t) * _K
                pltpu.sync_copy(es_hbm.at[pl.ds(goff, _K)], es_v)
                pltpu.sync_copy(ed_hbm.at[pl.ds(goff, _K)], ed_v)
                for k16 in range(_K // 16):
                    sl = pl.ds(k16 * 16, 16)
                    idx_v[sl] = es_v[sl] + p * (ns2 // 2)
                pltpu.async_copy(hs_hbm.at[idx_v], rows, sem1).wait()
                pltpu.sync_copy(ex_hbm.at[pl.ds(goff, _K)], exc)
                h0 = jnp.zeros((16,), jnp.int32) + 2 * p
                h1 = h0 + 1
                for kk in range(_K):
                    e0 = _bcast16(exc[kk, :], h0)
                    e1 = _bcast16(exc[kk, :], h1)
                    for cc in range(4):
                        sl = pl.ds(cc * 16, 16)
                        rows[kk, sl] = rows[kk, sl] * e0
                    for cc in range(4, 8):
                        sl = pl.ds(cc * 16, 16)
                        rows[kk, sl] = rows[kk, sl] * e1
                if half:
                    for k16 in range(_K // 16):
                        sl = pl.ds(k16 * 16, 16)
                        dv = ed_v[sl] - base
                        ok = (dv >= 0) & (dv < QR)
                        idx_v[sl] = jnp.where(ok, dv, QR)
                    pltpu.sync_copy(rows, acc_sh.at[idx_v], add=True)
                else:
                    pltpu.sync_copy(rows, acc_sh.at[ed_v], add=True)
                return 0
            lax.fori_loop(0, n_ch, body, 0)
            plsc.subcore_barrier()

            for j in range(m):
                off = (s * m + j) * chunk
                pltpu.sync_copy(acc_sh.at[pl.ds(off, chunk)], bo)
                if half:
                    pltpu.sync_copy(bo, ac_hbm.at[p, pl.ds(base + off, chunk)])
                else:
                    pltpu.sync_copy(bo, ac_hbm.at[c, p, pl.ds(off, chunk)])
            plsc.subcore_barrier()
            return 0
        lax.fori_loop(0, n_pass, pass_body, 0)
    return kern


@functools.lru_cache(maxsize=None)
def _sc_b2(E_pad, ns, nd, CO, half):
    if half:
        OWN, chunk, m = _region(-(-nd // 8))
        n_ch = E_pad // (16 * _K)
        out_ty = jax.ShapeDtypeStruct((8 * OWN, 128), jnp.float32)
        n_pass = 4
    else:
        OWN, chunk, m = _region(nd + 1)
        n_ch = E_pad // (32 * _K)
        out_ty = jax.ShapeDtypeStruct((2, OWN, 128), jnp.float32)
        n_pass = 1
    mesh = plsc.VectorSubcoreMesh(**_MESH)

    @functools.partial(
        pl.kernel, mesh=mesh,
        out_type=out_ty,
        scratch_types=[
            pltpu.VMEM((_K,), jnp.int32),
            pltpu.VMEM((_K,), jnp.int32),
            pltpu.VMEM((_K,), jnp.int32),
            pltpu.VMEM((_K, 16), jnp.float32),
            pltpu.VMEM((_K, 128), jnp.float32),
            pltpu.VMEM((_K, 128), jnp.float32),
            pltpu.VMEM((chunk, 128), jnp.float32),
            pltpu.VMEM((chunk, 128), jnp.float32),
            pltpu.VMEM_SHARED((OWN + 8, 128), jnp.float32),
            pltpu.SemaphoreType.DMA,
        ])
    def kern(es_hbm, ed_hbm, ex_hbm, hs_hbm, ac_hbm,
             es_v, ed_v, idx_v, exc, rows, sc_rows, zb, bo, acc_sh, sem1):
        c = lax.axis_index("c")
        s = lax.axis_index("s")
        _zero_fill(zb, chunk, 128)
        h0 = jnp.zeros((16,), jnp.int32)

        def pass_body(ps, _):
            base = (c * 4 + ps) * OWN if half else 0
            for j in range(m):
                pltpu.sync_copy(zb, acc_sh.at[pl.ds((s * m + j) * chunk, chunk)])
            plsc.subcore_barrier()

            def body(t, _):
                if half:
                    goff = (s * n_ch + t) * _K
                else:
                    goff = ((c * 16 + s) * n_ch + t) * _K
                pltpu.sync_copy(es_hbm.at[pl.ds(goff, _K)], es_v)
                pltpu.sync_copy(ed_hbm.at[pl.ds(goff, _K)], ed_v)
                pltpu.async_copy(hs_hbm.at[es_v], rows, sem1).wait()
                pltpu.sync_copy(ex_hbm.at[pl.ds(goff, _K)], exc)
                for kk in range(_K):
                    e0 = _bcast16(exc[kk, :], h0)
                    for cc in range(OUT // 16):
                        sl = pl.ds(CO + cc * 16, 16)
                        sc_rows[kk, sl] = rows[kk, sl] * e0
                if half:
                    for k16 in range(_K // 16):
                        sl = pl.ds(k16 * 16, 16)
                        dv = ed_v[sl] - base
                        ok = (dv >= 0) & (dv < OWN)
                        idx_v[sl] = jnp.where(ok, dv, OWN)
                    pltpu.sync_copy(sc_rows, acc_sh.at[idx_v], add=True)
                else:
                    pltpu.sync_copy(sc_rows, acc_sh.at[ed_v], add=True)
                return 0
            lax.fori_loop(0, n_ch, body, 0)
            plsc.subcore_barrier()

            for j in range(m):
                off = (s * m + j) * chunk
                pltpu.sync_copy(acc_sh.at[pl.ds(off, chunk)], bo)
                if half:
                    pltpu.sync_copy(bo, ac_hbm.at[pl.ds(base + off, chunk)])
                else:
                    pltpu.sync_copy(bo, ac_hbm.at[c, pl.ds(off, chunk)])
            plsc.subcore_barrier()
            return 0
        lax.fori_loop(0, n_pass, pass_body, 0)
    return kern


def _after(token, *xs):
    out = lax.optimization_barrier(tuple(xs) + (token,))
    return out[:-1]


def _pad_edges(edge, E_pad, nd):
    E = edge.shape[1]
    es = jnp.concatenate([edge[0], jnp.zeros((E_pad - E,), edge.dtype)])
    ed = jnp.concatenate([edge[1], jnp.full((E_pad - E,), nd, edge.dtype)])
    return es, ed


def _epad(E):
    return -(-E // 4096) * 4096


def _veff(W, att, heads, outc):
    W3 = W.reshape(W.shape[0], heads, outc)
    V = (W3 * att[0][None, :, :]).sum(-1)
    return jnp.pad(V, ((0, 0), (0, 16 - heads)))


def kernel(x_user, x_product, x_category, x_style, x_brand, x_color, edge_similar_to, edge_belongs_to, edge_has_style, edge_prefers, edge_viewed, proj_W_user, proj_b_user, proj_W_product, proj_b_product, proj_W_category, proj_b_category, proj_W_style, proj_b_style, proj_W_brand, proj_b_brand, proj_W_color, proj_b_color, c1_similar_to_W, c1_similar_to_as, c1_similar_to_ad, c1_similar_to_b, c1_belongs_to_W, c1_belongs_to_as, c1_belongs_to_ad, c1_belongs_to_b, c1_has_style_W, c1_has_style_as, c1_has_style_ad, c1_has_style_b, c1_prefers_W, c1_prefers_as, c1_prefers_ad, c1_prefers_b, c1_viewed_W, c1_viewed_as, c1_viewed_ad, c1_viewed_b, c2_similar_to_W, c2_similar_to_as, c2_similar_to_ad, c2_similar_to_b, c2_belongs_to_W, c2_belongs_to_as, c2_belongs_to_ad, c2_belongs_to_b, c2_prefers_W, c2_prefers_as, c2_prefers_ad, c2_prefers_b, out_W_product, out_b_product, out_W_category, out_b_category):
    np_, nu, nc = x_product.shape[0], x_user.shape[0], x_category.shape[0]

    Aw_p = jnp.concatenate([
        _veff(c1_similar_to_W, c1_similar_to_as, HEADS, HID),
        _veff(c1_similar_to_W, c1_similar_to_ad, HEADS, HID),
        _veff(c1_belongs_to_W, c1_belongs_to_as, HEADS, HID),
        _veff(c1_prefers_W, c1_prefers_ad, HEADS, HID),
        _veff(c1_viewed_W, c1_viewed_ad, HEADS, HID),
    ], axis=1)
    Aw_u = jnp.concatenate([
        _veff(c1_prefers_W, c1_prefers_as, HEADS, HID),
        _veff(c1_viewed_W, c1_viewed_as, HEADS, HID),
    ], axis=1)
    A2w_p = jnp.concatenate([
        _veff(c2_similar_to_W, c2_similar_to_as, 1, OUT),
        _veff(c2_similar_to_W, c2_similar_to_ad, 1, OUT),
        _veff(c2_belongs_to_W, c2_belongs_to_as, 1, OUT),
    ], axis=1)
    A2w_c = _veff(c2_belongs_to_W, c2_belongs_to_ad, 1, OUT)

    (sim_a, sim_b2, bel_a, bel_b2, hs_sim_b, exsl_sim, tab_p) = _pre(
        x_product, proj_W_product, proj_b_product,
        c1_similar_to_W, c1_belongs_to_W, Aw_p, 2000)
    (pre_a, pre_b2, vie_a, vie_b2, _, _, tab_u) = _pre(
        x_user, proj_W_user, proj_b_user,
        c1_prefers_W, c1_viewed_W, Aw_u, 2000)
    Aw_c = _veff(c1_belongs_to_W, c1_belongs_to_ad, HEADS, HID)
    _, _, _, _, _, _, tab_c = _pre(x_category, proj_W_category, proj_b_category,
                                   Aw_c, Aw_c, Aw_c, 1000)
    hs_sim_t = jnp.concatenate([sim_a, sim_b2], axis=0)
    hs_bel_t = jnp.concatenate([bel_a, bel_b2], axis=0)
    hs_pre_t = jnp.concatenate([pre_a, pre_b2], axis=0)
    hs_vie_t = jnp.concatenate([vie_a, vie_b2], axis=0)

    hs_sim3 = hs_sim_b.reshape(np_, HEADS, HID)

    EsP = _epad(edge_similar_to.shape[1])
    EbP = _epad(edge_belongs_to.shape[1])
    EpP = _epad(edge_prefers.shape[1])
    EvP = _epad(edge_viewed.shape[1])
    es_s, ed_s = _pad_edges(edge_similar_to, EsP, np_)
    es_b, ed_b = _pad_edges(edge_belongs_to, EbP, nc)
    es_p, ed_p = _pad_edges(edge_prefers, EpP, np_)
    es_v, ed_v = _pad_edges(edge_viewed, EvP, np_)

    ex_s = _sc_stage_a(EsP, np_, np_, 0, 16)(es_s, ed_s, tab_p, tab_p)
    es_b, ed_b = _after(ex_s[0], es_b, ed_b)
    ex_b = _sc_stage_a(EbP, np_, nc, 32, 0)(es_b, ed_b, tab_p, tab_c)
    es_p, ed_p = _after(ex_b[0], es_p, ed_p)
    ex_p = _sc_stage_a(EpP, nu, np_, 0, 48)(es_p, ed_p, tab_u, tab_p)
    es_v, ed_v = _after(ex_p[0], es_v, ed_v)
    ex_v = _sc_stage_a(EvP, nu, np_, 16, 64)(es_v, ed_v, tab_u, tab_p)

    ed_s2, = _after(ex_v[0], ed_s)
    dns = _sc_den(EsP, np_, True)(es_s, ed_s2, ex_s)
    ed_b2, = _after(dns[0], ed_b)
    dnb2 = _sc_den(EbP, nc, False)(es_b, ed_b2, ex_b)
    ed_p2, = _after(dnb2[0, 0], ed_p)
    dnp = _sc_den(EpP, np_, True)(es_p, ed_p2, ex_p)
    ed_v2, = _after(dnp[0], ed_v)
    dnv = _sc_den(EvP, np_, True)(es_v, ed_v2, ex_v)
    dnb = dnb2[0] + dnb2[1]

    es_s, ed_s = _after(dnv[0], es_s, ed_s)
    acs = _sc_b1(EsP, 2 * np_, np_, True)(es_s, ed_s, ex_s, hs_sim_t)
    es_b, ed_b = _after(acs[0, 0], es_b, ed_b)
    acb2 = _sc_b1(EbP, 2 * np_, nc, False)(es_b, ed_b, ex_b, hs_bel_t)
    es_p, ed_p = _after(acb2[0, 0, 0], es_p, ed_p)
    acp = _sc_b1(EpP, 2 * nu, np_, True)(es_p, ed_p, ex_p, hs_pre_t)
    es_v, ed_v = _after(acp[0, 0], es_v, ed_v)
    acv = _sc_b1(EvP, 2 * nu, np_, True)(es_v, ed_v, ex_v, hs_vie_t)

    acb = acb2[0] + acb2[1]

    hs2s_b, exsl2, tab2, a2tab = _mid_product(
        acs, dns, exsl_sim, hs_sim3, acp, dnp, acv, dnv,
        c1_similar_to_b, c1_prefers_b, c1_viewed_b,
        c2_similar_to_W, c2_belongs_to_W, A2w_p, np_, 2000)
    a2tab_c = _mid_category(acb, dnb, c1_belongs_to_b, A2w_c, nc)

    es_s, ed_s = _after(a2tab[0], es_s, ed_s)
    ex2s = _sc_stage_a(EsP, np_, np_, 0, 16)(es_s, ed_s, a2tab, a2tab)
    es_b, ed_b = _after(ex2s[0], es_b, ed_b)
    ex2b = _sc_stage_a(EbP, np_, nc, 32, 0)(es_b, ed_b, a2tab, a2tab_c)
    ed_s3, = _after(ex2b[0], ed_s)
    dn2s = _sc_den(EsP, np_, True)(es_s, ed_s3, ex2s)
    ed_b3, = _after(dn2s[0], ed_b)
    dn2b2 = _sc_den(EbP, nc, False)(es_b, ed_b3, ex2b)
    es_s, ed_s = _after(dn2b2[0, 0], es_s, ed_s)
    ac2s = _sc_b2(EsP, np_, np_, 0, True)(es_s, ed_s, ex2s, tab2)
    es_b, ed_b = _after(ac2s[0], es_b, ed_b)
    ac2b2 = _sc_b2(EbP, np_, nc, OUT, False)(es_b, ed_b, ex2b, tab2)
    dn2b = dn2b2[0] + dn2b2[1]
    ac2b = ac2b2[0] + ac2b2[1]

    out_p = _final_product(ac2s, dn2s, exsl2, hs2s_b,
                           c2_similar_to_b, out_W_product, out_b_product, np_, 2000)
    out_c = _final_category(ac2b, dn2b, c2_belongs_to_b,
                            out_W_category, out_b_category, nc)
    return (out_p, out_c)
